# baseline (speedup 1.0000x reference)
import functools

import jax
import jax.numpy as jnp
import numpy as np
from jax import lax
from jax.experimental import pallas as pl
from jax.experimental.pallas import tpu as pltpu
from jax.experimental.pallas import tpu_sc as plsc

H = 256
ATOM_FDIM = 128
BOND_FDIM = 144
N_ATOMS = 10000
N_BONDS = 320000
MAX_NB = 32
N_MOLS = 500
ATOMS_PER_MOL = 20
DEPTH = 3

NC, NS = 2, 16
NW = NC * NS
NA_PAD = 10240
A_W = NA_PAD // NW
A_CHUNK = 2
A_NBUF = 6
A_NCHUNK = A_W // A_CHUNK
B_W = N_BONDS // NW
B_CHUNK = 40
B_NCHUNK = B_W // B_CHUNK
B_NBUF = 4

BF = jnp.bfloat16

_mesh = functools.partial(
    plsc.VectorSubcoreMesh,
    core_axis_name="c", subcore_axis_name="s", num_cores=NC, num_subcores=NS)


def _wid():
    return lax.axis_index("s") * NC + lax.axis_index("c")


NIDX = A_CHUNK * MAX_NB


@functools.partial(
    pl.kernel,
    out_type=jax.ShapeDtypeStruct((NA_PAD, H), jnp.float32),
    mesh=_mesh(),
    scratch_types=[
        pltpu.VMEM((A_W * MAX_NB,), jnp.int32),
        [pltpu.VMEM((NIDX, H), jnp.float32) for _ in range(A_NBUF)],
        [pltpu.VMEM((A_CHUNK, H), jnp.float32) for _ in range(A_NBUF)],
        [pltpu.SemaphoreType.DMA for _ in range(A_NBUF)],
        [pltpu.SemaphoreType.DMA for _ in range(A_NBUF)],
    ],
)
def _sc_segsum(msg_hbm, a2b_hbm, am_hbm, idx_v, rows_v, out_v, gsem, wsem):
    w = _wid()
    nk = H // 16

    pltpu.sync_copy(a2b_hbm.at[pl.ds(w * A_W * MAX_NB, A_W * MAX_NB)], idx_v)

    def gather(c, b):
        pltpu.async_copy(
            msg_hbm.at[idx_v.at[pl.ds(c * NIDX, NIDX)]], rows_v[b], gsem[b])

    for b in range(A_NBUF - 1):
        gather(b, b)

    def chunk(c, b):
        pltpu.make_async_copy(
            msg_hbm.at[idx_v.at[pl.ds(c * NIDX, NIDX)]],
            rows_v[b], gsem[b]).wait()

        @pl.when(c + A_NBUF - 1 < A_NCHUNK)
        def _():
            gather(c + A_NBUF - 1, (b + A_NBUF - 1) % A_NBUF)

        base = w * A_W + c * A_CHUNK

        @pl.when(c >= A_NBUF)
        def _():
            pltpu.make_async_copy(
                out_v[b],
                am_hbm.at[pl.ds(base - A_NBUF * A_CHUNK, A_CHUNK)],
                wsem[b]).wait()

        for a in range(A_CHUNK):
            def row2(r, acc):
                i = a * MAX_NB + 2 * r
                v0 = [rows_v[b][i, pl.ds(k * 16, 16)] for k in range(nk)]
                v1 = [rows_v[b][i + 1, pl.ds(k * 16, 16)] for k in range(nk)]
                return tuple(acc[k] + (v0[k] + v1[k]) for k in range(nk))
            acc = lax.fori_loop(
                0, MAX_NB // 2, row2,
                tuple(jnp.zeros((16,), jnp.float32) for _ in range(nk)))
            for k in range(nk):
                out_v[b][a, pl.ds(k * 16, 16)] = acc[k]

        pltpu.async_copy(out_v[b], am_hbm.at[pl.ds(base, A_CHUNK)], wsem[b])

    def group(g, _):
        for b in range(A_NBUF):
            chunk(A_NBUF * g + b, b)
        return 0

    angroups = A_NCHUNK // A_NBUF
    lax.fori_loop(0, angroups, group, 0)
    for j in range(A_NCHUNK - angroups * A_NBUF):
        c = angroups * A_NBUF + j
        chunk(c, c % A_NBUF)
    for j in range(A_NBUF):
        c = A_NCHUNK - A_NBUF + j
        base = w * A_W + c * A_CHUNK
        pltpu.make_async_copy(
            out_v[c % A_NBUF],
            am_hbm.at[pl.ds(base, A_CHUNK)], wsem[c % A_NBUF]).wait()


@functools.partial(
    pl.kernel,
    out_type=jax.ShapeDtypeStruct((N_BONDS, H), jnp.float32),
    mesh=_mesh(),
    scratch_types=[
        pltpu.VMEM((B_W,), jnp.int32),
        pltpu.VMEM((B_W,), jnp.int32),
        [pltpu.VMEM((B_CHUNK, H), jnp.float32) for _ in range(B_NBUF)],
        [pltpu.VMEM((B_CHUNK, H), jnp.float32) for _ in range(B_NBUF)],
        [pltpu.VMEM((B_CHUNK, H), jnp.float32) for _ in range(2)],
        [pltpu.SemaphoreType.DMA for _ in range(B_NBUF)],
        [pltpu.SemaphoreType.DMA for _ in range(B_NBUF)],
        [pltpu.SemaphoreType.DMA for _ in range(2)],
    ],
)
def _sc_combine(msg_hbm, am_hbm, b2a_hbm, b2revb_hbm, t_hbm,
                idxa_v, idxr_v, am_v, rev_v, out_v, sema, semr, semw):
    w = _wid()

    def gathers(c, b):
        pltpu.async_copy(
            am_hbm.at[idxa_v.at[pl.ds(c * B_CHUNK, B_CHUNK)]], am_v[b],
            sema[b])
        pltpu.async_copy(
            msg_hbm.at[idxr_v.at[pl.ds(c * B_CHUNK, B_CHUNK)]], rev_v[b],
            semr[b])

    pltpu.sync_copy(b2a_hbm.at[pl.ds(w * B_W, B_W)], idxa_v)
    pltpu.sync_copy(b2revb_hbm.at[pl.ds(w * B_W, B_W)], idxr_v)
    for b in range(B_NBUF - 1):
        gathers(b, b)

    def chunk(c, b):
        ob = b % 2
        pltpu.make_async_copy(
            am_hbm.at[idxa_v.at[pl.ds(c * B_CHUNK, B_CHUNK)]], am_v[b],
            sema[b]).wait()
        pltpu.make_async_copy(
            msg_hbm.at[idxr_v.at[pl.ds(c * B_CHUNK, B_CHUNK)]], rev_v[b],
            semr[b]).wait()

        @pl.when(c + B_NBUF - 1 < B_NCHUNK)
        def _():
            gathers(c + B_NBUF - 1, (b + B_NBUF - 1) % B_NBUF)

        base = w * B_W + c * B_CHUNK

        @pl.when(c >= 2)
        def _():
            pltpu.make_async_copy(
                out_v[ob],
                t_hbm.at[pl.ds(base - 2 * B_CHUNK, B_CHUNK)], semw[ob]).wait()

        def row(r, _):
            nk = H // 16
            av = [am_v[b][r, pl.ds(k * 16, 16)] for k in range(nk)]
            rv = [rev_v[b][r, pl.ds(k * 16, 16)] for k in range(nk)]
            for k in range(nk):
                out_v[ob][r, pl.ds(k * 16, 16)] = av[k] - rv[k]
            return 0

        lax.fori_loop(0, B_CHUNK, row, 0)
        pltpu.async_copy(out_v[ob], t_hbm.at[pl.ds(base, B_CHUNK)], semw[ob])

    def group(g, _):
        for b in range(B_NBUF):
            chunk(B_NBUF * g + b, b)
        return 0

    ngroups = B_NCHUNK // B_NBUF
    lax.fori_loop(0, ngroups, group, 0)
    for j in range(B_NCHUNK - ngroups * B_NBUF):
        c = ngroups * B_NBUF + j
        chunk(c, c % B_NBUF)
    for c in (B_NCHUNK - 2, B_NCHUNK - 1):
        base = w * B_W + c * B_CHUNK
        pltpu.make_async_copy(
            out_v[c % 2], t_hbm.at[pl.ds(base, B_CHUNK)], semw[c % 2]).wait()


BM = 2000


def _in_proj_body(x_ref, w_ref, inp_ref, msg_ref):
    y = jnp.dot(x_ref[...], w_ref[...], preferred_element_type=jnp.float32)
    inp_ref[...] = y.astype(BF)
    msg_ref[...] = jnp.maximum(y, 0.0)


def _tc_in_proj(f_bonds, w_t):
    return pl.pallas_call(
        _in_proj_body,
        grid=(N_BONDS // BM,),
        compiler_params=pltpu.CompilerParams(
            allow_input_fusion=[True, False]),
        in_specs=[
            pl.BlockSpec((BM, BOND_FDIM), lambda i: (i, 0)),
            pl.BlockSpec((BOND_FDIM, H), lambda i: (0, 0)),
        ],
        out_specs=[
            pl.BlockSpec((BM, H), lambda i: (i, 0)),
            pl.BlockSpec((BM, H), lambda i: (i, 0)),
        ],
        out_shape=[
            jax.ShapeDtypeStruct((N_BONDS, H), BF),
            jax.ShapeDtypeStruct((N_BONDS, H), jnp.float32),
        ],
    )(f_bonds, w_t)


def _update_body(x_ref, w_ref, b_ref, o_ref):
    y = b_ref[...].astype(jnp.float32) + jnp.dot(
        x_ref[...].astype(BF), w_ref[...],
        preferred_element_type=jnp.float32)
    o_ref[...] = jnp.maximum(y, 0.0)


def _tc_update(t, wh_t, inp):
    return pl.pallas_call(
        _update_body,
        grid=(N_BONDS // BM,),
        in_specs=[
            pl.BlockSpec((BM, H), lambda i: (i, 0)),
            pl.BlockSpec((H, H), lambda i: (0, 0)),
            pl.BlockSpec((BM, H), lambda i: (i, 0)),
        ],
        out_specs=pl.BlockSpec((BM, H), lambda i: (i, 0)),
        out_shape=jax.ShapeDtypeStruct((N_BONDS, H), jnp.float32),
    )(t, wh_t, inp)


def _readout_body(fa_ref, am_ref, wo1_ref, wo2_ref, bo_ref, r0_ref, o_ref):
    hid = jnp.dot(fa_ref[...].astype(BF), wo1_ref[...],
                  preferred_element_type=jnp.float32)
    hid += jnp.dot(am_ref[:N_ATOMS].astype(BF), wo2_ref[...],
                   preferred_element_type=jnp.float32)
    hid = jnp.maximum(hid + bo_ref[...], 0.0)
    o_ref[...] = jnp.dot(r0_ref[...], hid, preferred_element_type=jnp.float32)


def _tc_readout(f_atoms, am, wo1_t, wo2_t, b_o2d, r0):
    return pl.pallas_call(
        _readout_body,
        out_shape=jax.ShapeDtypeStruct((N_MOLS, H), jnp.float32),
    )(f_atoms, am, wo1_t, wo2_t, b_o2d, r0)


_PAD_IDX = np.asarray(
    (np.arange((NA_PAD - N_ATOMS) * MAX_NB, dtype=np.int64) * 997)
    % N_BONDS, dtype=np.int32)
_R0 = np.kron(np.eye(N_MOLS, dtype=np.float32),
              np.full((1, ATOMS_PER_MOL), 1.0 / ATOMS_PER_MOL, np.float32))


def kernel(f_atoms, f_bonds, a2b, b2a, b2revb, W_i, W_h, W_o, b_o):
    a2b_flat = jnp.concatenate([a2b.reshape(-1), jnp.asarray(_PAD_IDX)])
    wi_t = W_i.T.astype(BF)
    wh_t = W_h.T.astype(BF)
    wo1_t = W_o[:, :ATOM_FDIM].T.astype(BF)
    wo2_t = W_o[:, ATOM_FDIM:].T.astype(BF)
    b_o2d = b_o.reshape(1, H)

    inp, msg = _tc_in_proj(f_bonds.astype(BF), wi_t)
    for _ in range(DEPTH - 1):
        am = _sc_segsum(msg, a2b_flat)
        t = _sc_combine(msg, am, b2a, b2revb)
        msg = _tc_update(t, wh_t, inp)
    am = _sc_segsum(msg, a2b_flat)
    return _tc_readout(f_atoms, am, wo1_t, wo2_t, b_o2d, jnp.asarray(_R0))


# speedup vs baseline: 1.0097x; 1.0097x over previous
import functools

import jax
import jax.numpy as jnp
import numpy as np
from jax import lax
from jax.experimental import pallas as pl
from jax.experimental.pallas import tpu as pltpu
from jax.experimental.pallas import tpu_sc as plsc

H = 256
ATOM_FDIM = 128
BOND_FDIM = 144
N_ATOMS = 10000
N_BONDS = 320000
MAX_NB = 32
N_MOLS = 500
ATOMS_PER_MOL = 20
DEPTH = 3

NC, NS = 2, 16
NW = NC * NS
NA_PAD = 10240
A_W = NA_PAD // NW
A_CHUNK = 2
A_NBUF = 5
A_NCHUNK = A_W // A_CHUNK
B_W = N_BONDS // NW
B_CHUNK = 40
B_NCHUNK = B_W // B_CHUNK
B_NBUF = 4

BF = jnp.bfloat16

_mesh = functools.partial(
    plsc.VectorSubcoreMesh,
    core_axis_name="c", subcore_axis_name="s", num_cores=NC, num_subcores=NS)


def _wid():
    return lax.axis_index("s") * NC + lax.axis_index("c")


NIDX = A_CHUNK * MAX_NB


@functools.partial(
    pl.kernel,
    out_type=jax.ShapeDtypeStruct((NA_PAD, H), jnp.float32),
    mesh=_mesh(),
    scratch_types=[
        pltpu.VMEM((A_W * MAX_NB,), jnp.int32),
        [pltpu.VMEM((NIDX, H), jnp.float32) for _ in range(A_NBUF)],
        [pltpu.VMEM((A_CHUNK, H), jnp.float32) for _ in range(A_NBUF)],
        [pltpu.SemaphoreType.DMA for _ in range(A_NBUF)],
        [pltpu.SemaphoreType.DMA for _ in range(A_NBUF)],
    ],
)
def _sc_segsum(msg_hbm, a2b_hbm, am_hbm, idx_v, rows_v, out_v, gsem, wsem):
    w = _wid()
    nk = H // 16

    pltpu.sync_copy(a2b_hbm.at[pl.ds(w * A_W * MAX_NB, A_W * MAX_NB)], idx_v)

    def gather(c, b):
        pltpu.async_copy(
            msg_hbm.at[idx_v.at[pl.ds(c * NIDX, NIDX)]], rows_v[b], gsem[b])

    for b in range(A_NBUF - 1):
        gather(b, b)

    def chunk(c, b):
        pltpu.make_async_copy(
            msg_hbm.at[idx_v.at[pl.ds(c * NIDX, NIDX)]],
            rows_v[b], gsem[b]).wait()

        @pl.when(c + A_NBUF - 1 < A_NCHUNK)
        def _():
            gather(c + A_NBUF - 1, (b + A_NBUF - 1) % A_NBUF)

        base = w * A_W + c * A_CHUNK

        @pl.when(c >= A_NBUF)
        def _():
            pltpu.make_async_copy(
                out_v[b],
                am_hbm.at[pl.ds(base - A_NBUF * A_CHUNK, A_CHUNK)],
                wsem[b]).wait()

        for a in range(A_CHUNK):
            def row2(r, acc):
                i = a * MAX_NB + 2 * r
                v0 = [rows_v[b][i, pl.ds(k * 16, 16)] for k in range(nk)]
                v1 = [rows_v[b][i + 1, pl.ds(k * 16, 16)] for k in range(nk)]
                return tuple(acc[k] + (v0[k] + v1[k]) for k in range(nk))
            acc = lax.fori_loop(
                0, MAX_NB // 2, row2,
                tuple(jnp.zeros((16,), jnp.float32) for _ in range(nk)))
            for k in range(nk):
                out_v[b][a, pl.ds(k * 16, 16)] = acc[k]

        pltpu.async_copy(out_v[b], am_hbm.at[pl.ds(base, A_CHUNK)], wsem[b])

    def group(g, _):
        for b in range(A_NBUF):
            chunk(A_NBUF * g + b, b)
        return 0

    angroups = A_NCHUNK // A_NBUF
    lax.fori_loop(0, angroups, group, 0)
    for j in range(A_NCHUNK - angroups * A_NBUF):
        c = angroups * A_NBUF + j
        chunk(c, c % A_NBUF)
    for j in range(A_NBUF):
        c = A_NCHUNK - A_NBUF + j
        base = w * A_W + c * A_CHUNK
        pltpu.make_async_copy(
            out_v[c % A_NBUF],
            am_hbm.at[pl.ds(base, A_CHUNK)], wsem[c % A_NBUF]).wait()


@functools.partial(
    pl.kernel,
    out_type=jax.ShapeDtypeStruct((N_BONDS, H), jnp.float32),
    mesh=_mesh(),
    scratch_types=[
        pltpu.VMEM((B_W,), jnp.int32),
        pltpu.VMEM((B_W,), jnp.int32),
        [pltpu.VMEM((B_CHUNK, H), jnp.float32) for _ in range(B_NBUF)],
        [pltpu.VMEM((B_CHUNK, H), jnp.float32) for _ in range(B_NBUF)],
        [pltpu.VMEM((B_CHUNK, H), jnp.float32) for _ in range(2)],
        [pltpu.SemaphoreType.DMA for _ in range(B_NBUF)],
        [pltpu.SemaphoreType.DMA for _ in range(B_NBUF)],
        [pltpu.SemaphoreType.DMA for _ in range(2)],
    ],
)
def _sc_combine(msg_hbm, am_hbm, b2a_hbm, b2revb_hbm, t_hbm,
                idxa_v, idxr_v, am_v, rev_v, out_v, sema, semr, semw):
    w = _wid()

    def gathers(c, b):
        pltpu.async_copy(
            am_hbm.at[idxa_v.at[pl.ds(c * B_CHUNK, B_CHUNK)]], am_v[b],
            sema[b])
        pltpu.async_copy(
            msg_hbm.at[idxr_v.at[pl.ds(c * B_CHUNK, B_CHUNK)]], rev_v[b],
            semr[b])

    pltpu.sync_copy(b2a_hbm.at[pl.ds(w * B_W, B_W)], idxa_v)
    pltpu.sync_copy(b2revb_hbm.at[pl.ds(w * B_W, B_W)], idxr_v)
    for b in range(B_NBUF - 1):
        gathers(b, b)

    def chunk(c, b):
        ob = b % 2
        pltpu.make_async_copy(
            am_hbm.at[idxa_v.at[pl.ds(c * B_CHUNK, B_CHUNK)]], am_v[b],
            sema[b]).wait()
        pltpu.make_async_copy(
            msg_hbm.at[idxr_v.at[pl.ds(c * B_CHUNK, B_CHUNK)]], rev_v[b],
            semr[b]).wait()

        @pl.when(c + B_NBUF - 1 < B_NCHUNK)
        def _():
            gathers(c + B_NBUF - 1, (b + B_NBUF - 1) % B_NBUF)

        base = w * B_W + c * B_CHUNK

        @pl.when(c >= 2)
        def _():
            pltpu.make_async_copy(
                out_v[ob],
                t_hbm.at[pl.ds(base - 2 * B_CHUNK, B_CHUNK)], semw[ob]).wait()

        def row(r, _):
            nk = H // 16
            av = [am_v[b][r, pl.ds(k * 16, 16)] for k in range(nk)]
            rv = [rev_v[b][r, pl.ds(k * 16, 16)] for k in range(nk)]
            for k in range(nk):
                out_v[ob][r, pl.ds(k * 16, 16)] = av[k] - rv[k]
            return 0

        lax.fori_loop(0, B_CHUNK, row, 0)
        pltpu.async_copy(out_v[ob], t_hbm.at[pl.ds(base, B_CHUNK)], semw[ob])

    def group(g, _):
        for b in range(B_NBUF):
            chunk(B_NBUF * g + b, b)
        return 0

    ngroups = B_NCHUNK // B_NBUF
    lax.fori_loop(0, ngroups, group, 0)
    for j in range(B_NCHUNK - ngroups * B_NBUF):
        c = ngroups * B_NBUF + j
        chunk(c, c % B_NBUF)
    for c in (B_NCHUNK - 2, B_NCHUNK - 1):
        base = w * B_W + c * B_CHUNK
        pltpu.make_async_copy(
            out_v[c % 2], t_hbm.at[pl.ds(base, B_CHUNK)], semw[c % 2]).wait()


BM = 2000


def _in_proj_body(x_ref, w_ref, inp_ref, msg_ref):
    y = jnp.dot(x_ref[...], w_ref[...], preferred_element_type=jnp.float32)
    inp_ref[...] = y.astype(BF)
    msg_ref[...] = jnp.maximum(y, 0.0)


def _tc_in_proj(f_bonds, w_t):
    return pl.pallas_call(
        _in_proj_body,
        grid=(N_BONDS // BM,),
        compiler_params=pltpu.CompilerParams(
            allow_input_fusion=[True, False]),
        in_specs=[
            pl.BlockSpec((BM, BOND_FDIM), lambda i: (i, 0)),
            pl.BlockSpec((BOND_FDIM, H), lambda i: (0, 0)),
        ],
        out_specs=[
            pl.BlockSpec((BM, H), lambda i: (i, 0)),
            pl.BlockSpec((BM, H), lambda i: (i, 0)),
        ],
        out_shape=[
            jax.ShapeDtypeStruct((N_BONDS, H), BF),
            jax.ShapeDtypeStruct((N_BONDS, H), jnp.float32),
        ],
    )(f_bonds, w_t)


def _update_body(x_ref, w_ref, b_ref, o_ref):
    y = b_ref[...].astype(jnp.float32) + jnp.dot(
        x_ref[...].astype(BF), w_ref[...],
        preferred_element_type=jnp.float32)
    o_ref[...] = jnp.maximum(y, 0.0)


def _tc_update(t, wh_t, inp):
    return pl.pallas_call(
        _update_body,
        grid=(N_BONDS // BM,),
        in_specs=[
            pl.BlockSpec((BM, H), lambda i: (i, 0)),
            pl.BlockSpec((H, H), lambda i: (0, 0)),
            pl.BlockSpec((BM, H), lambda i: (i, 0)),
        ],
        out_specs=pl.BlockSpec((BM, H), lambda i: (i, 0)),
        out_shape=jax.ShapeDtypeStruct((N_BONDS, H), jnp.float32),
    )(t, wh_t, inp)


def _readout_body(fa_ref, am_ref, wo1_ref, wo2_ref, bo_ref, r0_ref, o_ref):
    hid = jnp.dot(fa_ref[...].astype(BF), wo1_ref[...],
                  preferred_element_type=jnp.float32)
    hid += jnp.dot(am_ref[:N_ATOMS].astype(BF), wo2_ref[...],
                   preferred_element_type=jnp.float32)
    hid = jnp.maximum(hid + bo_ref[...], 0.0)
    o_ref[...] = jnp.dot(r0_ref[...], hid, preferred_element_type=jnp.float32)


def _tc_readout(f_atoms, am, wo1_t, wo2_t, b_o2d, r0):
    return pl.pallas_call(
        _readout_body,
        out_shape=jax.ShapeDtypeStruct((N_MOLS, H), jnp.float32),
    )(f_atoms, am, wo1_t, wo2_t, b_o2d, r0)


_PAD_IDX = np.asarray(
    (np.arange((NA_PAD - N_ATOMS) * MAX_NB, dtype=np.int64) * 997)
    % N_BONDS, dtype=np.int32)
_R0 = np.kron(np.eye(N_MOLS, dtype=np.float32),
              np.full((1, ATOMS_PER_MOL), 1.0 / ATOMS_PER_MOL, np.float32))


def kernel(f_atoms, f_bonds, a2b, b2a, b2revb, W_i, W_h, W_o, b_o):
    a2b_flat = jnp.concatenate([a2b.reshape(-1), jnp.asarray(_PAD_IDX)])
    wi_t = W_i.T.astype(BF)
    wh_t = W_h.T.astype(BF)
    wo1_t = W_o[:, :ATOM_FDIM].T.astype(BF)
    wo2_t = W_o[:, ATOM_FDIM:].T.astype(BF)
    b_o2d = b_o.reshape(1, H)

    inp, msg = _tc_in_proj(f_bonds.astype(BF), wi_t)
    for _ in range(DEPTH - 1):
        am = _sc_segsum(msg, a2b_flat)
        t = _sc_combine(msg, am, b2a, b2revb)
        msg = _tc_update(t, wh_t, inp)
    am = _sc_segsum(msg, a2b_flat)
    return _tc_readout(f_atoms, am, wo1_t, wo2_t, b_o2d, jnp.asarray(_R0))
